# Initial kernel scaffold; baseline (speedup 1.0000x reference)
import jax
import jax.numpy as jnp
from jax.experimental import pallas as pl


def kernel(x_user, x_food, edge_index_eats, edge_index_rev, edge_label_index, W_proj_user, b_proj_user, W_proj_food, b_proj_food, lin_src_eats, lin_dst_eats, att_src_eats, att_dst_eats, bias_eats, lin_src_rev, lin_dst_rev, att_src_rev, att_dst_rev, bias_rev, gamma_user, beta_user, gamma_food, beta_food, W_dec1, b_dec1, W_dec2, b_dec2):
    raise NotImplementedError("write your pallas kernel here")


# speedup vs baseline: 18.1302x; 18.1302x over previous
import functools

import jax
import jax.numpy as jnp
from jax import lax
from jax.experimental import pallas as pl
from jax.experimental.pallas import tpu as pltpu
from jax.experimental.pallas import tpu_sc as plsc

N = 10000
NP = 10016
E = 320000
CCH = 64
ET = 20032
NCH = ET // CCH
EP = ET * 16
NLBL = 50000
CH = 128
LT = 1664
LP = LT * 32
PD = 160
NACC = 10048

_f32 = jnp.float32
_i32 = jnp.int32


def _k1_body(x_ref, w_ref, y_ref):
    y_ref[...] = jnp.dot(x_ref[...], w_ref[...], preferred_element_type=_f32)


def _k1(x, w):
    m, k = x.shape
    n = w.shape[1]
    blk = 1000
    return pl.pallas_call(
        _k1_body,
        grid=(m // blk,),
        in_specs=[
            pl.BlockSpec((blk, k), lambda i: (i, 0)),
            pl.BlockSpec((k, n), lambda i: (0, 0)),
        ],
        out_specs=pl.BlockSpec((blk, n), lambda i: (i, 0)),
        out_shape=jax.ShapeDtypeStruct((m, n), _f32),
    )(x, w)


def _cofa_body(asrc_h, adst_h, src_h, dst_h, gmax_h, coef_h,
               asrc_v, adst_v, part_v, srcb, dstb, cb0, cb1,
               tmp_v, gmax_v, denom_s):
    cid = lax.axis_index("c")
    sid = lax.axis_index("s")

    pltpu.sync_copy(asrc_h.at[pl.ds(cid * 2 * NP, 2 * NP)], asrc_v)
    pltpu.sync_copy(adst_h.at[pl.ds(cid * 2 * NP, 2 * NP)], adst_v)
    pltpu.sync_copy(gmax_h.at[pl.ds(cid * 32, 32)], gmax_v)

    def _initpart(i, _):
        part_v[i >> 3, pl.ds((i & 7) * 16, 16)] = jnp.zeros((16,), _f32)
        return 0
    lax.fori_loop(0, PD * 8, _initpart, 0)

    ebase = sid * ET

    def _alpha_ex(sv, dv, j):
        av = plsc.load_gather(asrc_v, [sv * 2 + j])
        bv = plsc.load_gather(adst_v, [dv * 2 + j])
        al = av + bv
        al = jnp.where(al >= 0.0, al, al * jnp.float32(0.2))
        return jnp.exp(al - gmax_v[pl.ds(j * 16, 16)])

    def _p1_chunk(ch, _):
        base = ebase + ch * CCH
        pltpu.sync_copy(src_h.at[pl.ds(base, CCH)], srcb)
        pltpu.sync_copy(dst_h.at[pl.ds(base, CCH)], dstb)

        lanes = lax.iota(_i32, 16)

        def _grp(k, _):
            sv = srcb[pl.ds(k * 16, 16)]
            dv = dstb[pl.ds(k * 16, 16)]
            ex0 = _alpha_ex(sv, dv, 0)
            ex1 = _alpha_ex(sv, dv, 1)
            f = dv * 2
            rv = f >> 7
            cbv = f & 112
            cv = f & 15
            for i in range(16):
                r = rv[i]
                cb = cbv[i]
                c = cv[i]
                row = part_v[r, pl.ds(cb, 16)]
                row = row + jnp.where(lanes == c, ex0[i], 0.0)
                row = row + jnp.where(lanes == c + 1, ex1[i], 0.0)
                part_v[r, pl.ds(cb, 16)] = row
            return 0
        lax.fori_loop(0, CCH // 16, _grp, 0)
        return 0
    lax.fori_loop(0, NCH, _p1_chunk, 0)

    pltpu.sync_copy(part_v, denom_s.at[pl.ds(sid * PD, PD)])
    plsc.subcore_barrier()

    def _csum(t, _):
        pltpu.sync_copy(denom_s.at[pl.ds(t * PD, PD)], tmp_v)

        def _acc(i, _):
            r = i >> 3
            cb = (i & 7) * 16
            part_v[r, pl.ds(cb, 16)] = (
                part_v[r, pl.ds(cb, 16)] + tmp_v[r, pl.ds(cb, 16)])
            return 0
        lax.fori_loop(0, PD * 8, _acc, 0)
        return 0

    def _czero(i, _):
        part_v[i >> 3, pl.ds((i & 7) * 16, 16)] = jnp.zeros((16,), _f32)
        return 0
    lax.fori_loop(0, PD * 8, _czero, 0)
    lax.fori_loop(0, 16, _csum, 0)

    def _co_chunk(ch, _):
        base = ebase + ch * CCH
        pltpu.sync_copy(src_h.at[pl.ds(base, CCH)], srcb)
        pltpu.sync_copy(dst_h.at[pl.ds(base, CCH)], dstb)

        def _grp(k, _):
            sv = srcb[pl.ds(k * 16, 16)]
            dv = dstb[pl.ds(k * 16, 16)]
            for j, cb in ((0, cb0), (1, cb1)):
                ex = _alpha_ex(sv, dv, j)
                f = dv * 2 + j
                den = plsc.load_gather(part_v, [f >> 7, f & 127])
                cb[pl.ds(k * 16, 16)] = ex / (den + jnp.float32(1e-16))
            return 0
        lax.fori_loop(0, CCH // 16, _grp, 0)
        pltpu.sync_copy(cb0, coef_h.at[pl.ds(2 * cid * EP + base, CCH)])
        pltpu.sync_copy(cb1,
                        coef_h.at[pl.ds((2 * cid + 1) * EP + base, CCH)])
        return 0
    lax.fori_loop(0, NCH, _co_chunk, 0)


_cofa_call = functools.partial(
    pl.kernel,
    out_type=jax.ShapeDtypeStruct((4 * EP,), _f32),
    compiler_params=pltpu.CompilerParams(needs_layout_passes=False),
    mesh=plsc.VectorSubcoreMesh(core_axis_name="c", subcore_axis_name="s",
                                num_cores=2, num_subcores=16),
    scratch_types=[
        pltpu.VMEM((2 * NP,), _f32),
        pltpu.VMEM((2 * NP,), _f32),
        pltpu.VMEM((PD, 128), _f32),
        pltpu.VMEM((CCH,), _i32),
        pltpu.VMEM((CCH,), _i32),
        pltpu.VMEM((CCH,), _f32),
        pltpu.VMEM((CCH,), _f32),
        pltpu.VMEM((PD, 128), _f32),
        pltpu.VMEM((32,), _f32),
        pltpu.VMEM_SHARED((16 * PD, 128), _f32),
    ],
)(_cofa_body)


def _msgb_body(src_h, dst_h, coef_h, hs_h, acc_h,
               rows_v, srcb, dstb, gidx, cb0, cb1, out_s, sem):
    cid = lax.axis_index("c")
    sid = lax.axis_index("s")
    ebase = sid * ET

    def _zr(i, _):
        rows_v[i >> 3, pl.ds((i & 7) * 16, 16)] = jnp.zeros((16,), _f32)
        return 0
    lax.fori_loop(0, CCH * 8, _zr, 0)

    def _zo(i, _):
        zcv = i * 16 + sid

        @pl.when(zcv < NACC // CCH)
        def _():
            pltpu.sync_copy(rows_v, out_s.at[pl.ds(zcv * CCH, CCH)])
        return 0
    lax.fori_loop(0, NACC // CCH // 16 + 1, _zo, 0)
    plsc.subcore_barrier()

    def _p2_chunk(ch, _):
        base = ebase + ch * CCH
        pltpu.sync_copy(src_h.at[pl.ds(base, CCH)], srcb)
        pltpu.sync_copy(dst_h.at[pl.ds(base, CCH)], dstb)
        pltpu.sync_copy(coef_h.at[pl.ds(2 * cid * EP + base, CCH)], cb0)
        pltpu.sync_copy(coef_h.at[pl.ds((2 * cid + 1) * EP + base, CCH)],
                        cb1)

        def _gi(k, _):
            gidx[pl.ds(k * 16, 16)] = srcb[pl.ds(k * 16, 16)] + cid * NP
            return 0
        lax.fori_loop(0, CCH // 16, _gi, 0)
        pltpu.async_copy(hs_h.at[gidx], rows_v, sem).wait()

        def _grp(k, _):
            cv0 = cb0[pl.ds(k * 16, 16)]
            cv1 = cb1[pl.ds(k * 16, 16)]
            for i in range(16):
                e = k * 16 + i
                c0 = cv0[i]
                c1 = cv1[i]
                for q in range(4):
                    rows_v[e, pl.ds(q * 16, 16)] = (
                        rows_v[e, pl.ds(q * 16, 16)] * c0)
                for q in range(4, 8):
                    rows_v[e, pl.ds(q * 16, 16)] = (
                        rows_v[e, pl.ds(q * 16, 16)] * c1)
            return 0
        lax.fori_loop(0, CCH // 16, _grp, 0)
        pltpu.sync_copy(rows_v, out_s.at[dstb], add=True)
        return 0
    lax.fori_loop(0, NCH, _p2_chunk, 0)
    plsc.subcore_barrier()

    @pl.when(sid < 10)
    def _():
        def _wr(i, _):
            r = sid * 1000 + i * CCH
            pltpu.sync_copy(out_s.at[pl.ds(r, CCH)], rows_v)
            pltpu.sync_copy(rows_v, acc_h.at[pl.ds(cid * NACC + r, CCH)])
            return 0
        lax.fori_loop(0, 1000 // CCH, _wr, 0)
        r = sid * 1000 + (1000 // CCH) * CCH
        nrem = 1000 - (1000 // CCH) * CCH
        if nrem:
            pltpu.sync_copy(out_s.at[pl.ds(r, nrem)],
                            rows_v.at[pl.ds(0, nrem)])
            pltpu.sync_copy(rows_v.at[pl.ds(0, nrem)],
                            acc_h.at[pl.ds(cid * NACC + r, nrem)])


_msgb_call = functools.partial(
    pl.kernel,
    out_type=jax.ShapeDtypeStruct((2 * NACC, 128), _f32),
    compiler_params=pltpu.CompilerParams(needs_layout_passes=False),
    mesh=plsc.VectorSubcoreMesh(core_axis_name="c", subcore_axis_name="s",
                                num_cores=2, num_subcores=16),
    scratch_types=[
        pltpu.VMEM((CCH, 128), _f32),
        pltpu.VMEM((CCH,), _i32),
        pltpu.VMEM((CCH,), _i32),
        pltpu.VMEM((CCH,), _i32),
        pltpu.VMEM((CCH,), _f32),
        pltpu.VMEM((CCH,), _f32),
        pltpu.VMEM_SHARED((NACC, 128), _f32),
        pltpu.SemaphoreType.DMA,
    ],
)(_msgb_body)


def _conv_call(asrc, adst, src, dst, hs, gmax):
    coef = _cofa_call(asrc, adst, src, dst, gmax)
    return _msgb_call(src, dst, coef, hs)


def _k2_body(ae_ref, ar_ref, w1_ref, prm_ref, uf_ref):
    prm = prm_ref[...]

    def head_mean(a, bias):
        return (a[0:N, 0:64] + a[0:N, 64:128]
                + a[NACC:NACC + N, 0:64]
                + a[NACC:NACC + N, 64:128]) * 0.25 + bias

    def bn_relu(x, g, b):
        mu = jnp.mean(x, axis=0, keepdims=True)
        var = jnp.mean((x - mu) * (x - mu), axis=0, keepdims=True)
        xn = (x - mu) * lax.rsqrt(var + 1e-5)
        return jnp.maximum(g * xn + b, 0.0)

    fo = head_mean(ae_ref[...], prm[0, 0:64])
    uo = head_mean(ar_ref[...], prm[1, 0:64])
    ur = bn_relu(uo, prm[2, 0:64], prm[3, 0:64])
    fr = bn_relu(fo, prm[4, 0:64], prm[5, 0:64])
    w1 = w1_ref[...]
    uf_ref[0:N, :] = (
        jnp.dot(ur, w1[0:64, :], preferred_element_type=_f32) + prm[6, :])
    uf_ref[N:2 * N, :] = jnp.dot(fr, w1[64:128, :],
                                 preferred_element_type=_f32)


def _k2(acc_e, acc_r, w1, prm):
    return pl.pallas_call(
        _k2_body,
        out_shape=jax.ShapeDtypeStruct((2 * N, 128), _f32),
    )(acc_e, acc_r, w1, prm)


def _dec_body(uf_h, idx_h, wb_h, out_h, i0b, i1b, urows, frows,
              lg, outb, wb_v, sem):
    cid = lax.axis_index("c")
    sid = lax.axis_index("s")
    wid = sid * 2 + cid
    pltpu.sync_copy(wb_h, wb_v)

    def _chunk(ch, _):
        base = wid * LT + ch * CH
        pltpu.sync_copy(idx_h.at[pl.ds(base, CH)], i0b)
        pltpu.sync_copy(idx_h.at[pl.ds(LP + base, CH)], i1b)
        pltpu.async_copy(uf_h.at[i0b], urows, sem).wait()
        pltpu.async_copy(uf_h.at[i1b], frows, sem).wait()
        lane0 = lax.iota(_i32, 16) == 0

        def _lab(e, _):
            v = jnp.zeros((16,), _f32)
            for k in range(8):
                h = urows[e, pl.ds(k * 16, 16)] + frows[e, pl.ds(k * 16, 16)]
                v = v + jnp.maximum(h, 0.0) * wb_v[pl.ds(k * 16, 16)]
            s = jnp.sum(v)
            plsc.store_scatter(lg, [jnp.full((16,), e, _i32)],
                               jnp.full((16,), s, _f32), mask=lane0)
            return 0
        lax.fori_loop(0, CH, _lab, 0)

        def _sg(k, _):
            v = lg[pl.ds(k * 16, 16)] + wb_v[pl.ds(128, 16)]
            outb[pl.ds(k * 16, 16)] = 1.0 / (1.0 + jnp.exp(-v))
            return 0
        lax.fori_loop(0, CH // 16, _sg, 0)
        pltpu.sync_copy(outb, out_h.at[pl.ds(base, CH)])
        return 0
    lax.fori_loop(0, LT // CH, _chunk, 0)


_dec_call = functools.partial(
    pl.kernel,
    out_type=jax.ShapeDtypeStruct((LP,), _f32),
    compiler_params=pltpu.CompilerParams(needs_layout_passes=False),
    mesh=plsc.VectorSubcoreMesh(core_axis_name="c", subcore_axis_name="s",
                                num_cores=2, num_subcores=16),
    scratch_types=[
        pltpu.VMEM((CH,), _i32),
        pltpu.VMEM((CH,), _i32),
        pltpu.VMEM((CH, 128), _f32),
        pltpu.VMEM((CH, 128), _f32),
        pltpu.VMEM((CH,), _f32),
        pltpu.VMEM((CH,), _f32),
        pltpu.VMEM((144,), _f32),
        pltpu.SemaphoreType.DMA,
    ],
)(_dec_body)


def _blockdiag_att(att):
    h, o = att.shape
    eye = jnp.eye(h, dtype=_f32)
    return (att[:, :, None] * eye[:, None, :]).reshape(h * o, h)


def _sc_tables(a):
    pad = jnp.zeros((NP - N, 4), _f32)
    ap = jnp.concatenate([a, pad], axis=0)
    per_core = ap.reshape(NP, 2, 2).transpose(1, 0, 2)
    return per_core.reshape(2 * 2 * NP)


def kernel(x_user, x_food, edge_index_eats, edge_index_rev, edge_label_index,
           W_proj_user, b_proj_user, W_proj_food, b_proj_food,
           lin_src_eats, lin_dst_eats, att_src_eats, att_dst_eats, bias_eats,
           lin_src_rev, lin_dst_rev, att_src_rev, att_dst_rev, bias_rev,
           gamma_user, beta_user, gamma_food, beta_food,
           W_dec1, b_dec1, W_dec2, b_dec2):
    f32 = _f32
    wu_se = W_proj_user @ lin_src_eats
    wu_dr = W_proj_user @ lin_dst_rev
    wf_sr = W_proj_food @ lin_src_rev
    wf_de = W_proj_food @ lin_dst_eats
    bu_se = b_proj_user @ lin_src_eats
    bu_dr = b_proj_user @ lin_dst_rev
    bf_sr = b_proj_food @ lin_src_rev
    bf_de = b_proj_food @ lin_dst_eats

    A_se = _blockdiag_att(att_src_eats)
    A_de = _blockdiag_att(att_dst_eats)
    A_sr = _blockdiag_att(att_src_rev)
    A_dr = _blockdiag_att(att_dst_rev)

    wbig_u = jnp.concatenate(
        [wu_se, wu_se @ A_se, wu_dr @ A_dr,
         jnp.zeros((128, 120), f32)], axis=1)
    wbig_f = jnp.concatenate(
        [wf_sr, wf_sr @ A_sr, wf_de @ A_de,
         jnp.zeros((128, 120), f32)], axis=1)

    yu = _k1(x_user, wbig_u)
    yf = _k1(x_food, wbig_f)

    hs_eats = yu[:, 0:256] + bu_se
    a_s_eats = yu[:, 256:260] + bu_se @ A_se
    a_d_rev = yu[:, 260:264] + bu_dr @ A_dr
    hs_rev = yf[:, 0:256] + bf_sr
    a_s_rev = yf[:, 256:260] + bf_sr @ A_sr
    a_d_eats = yf[:, 260:264] + bf_de @ A_de

    def gbound(a_s, a_d):
        m = jnp.max(a_s, axis=0) + jnp.max(a_d, axis=0)
        m = jnp.where(m >= 0, m, m * 0.2)
        return jnp.broadcast_to(
            m.reshape(2, 2, 1), (2, 2, 16)).astype(f32).reshape(64)

    g_eats = gbound(a_s_eats, a_d_eats)
    g_rev = gbound(a_s_rev, a_d_rev)

    def hs_pad(hs):
        z = jnp.zeros((NP - N, 128), f32)
        return jnp.concatenate([hs[:, 0:128], z, hs[:, 128:256], z], axis=0)

    def pad_edges(ei):
        padv = jnp.full((2, EP - E), N, jnp.int32)
        return jnp.concatenate([ei.astype(jnp.int32), padv], axis=1)

    ee = pad_edges(edge_index_eats)
    er = pad_edges(edge_index_rev)

    acc_eats = _conv_call(_sc_tables(a_s_eats), _sc_tables(a_d_eats),
                          ee[0], ee[1], hs_pad(hs_eats), g_eats)
    g_rev_seq, _ = lax.optimization_barrier((g_rev, acc_eats))
    acc_rev = _conv_call(_sc_tables(a_s_rev), _sc_tables(a_d_rev),
                         er[0], er[1], hs_pad(hs_rev), g_rev_seq)

    prm = jnp.zeros((8, 128), f32)
    prm = prm.at[0, 0:64].set(bias_eats)
    prm = prm.at[1, 0:64].set(bias_rev)
    prm = prm.at[2, 0:64].set(gamma_user)
    prm = prm.at[3, 0:64].set(beta_user)
    prm = prm.at[4, 0:64].set(gamma_food)
    prm = prm.at[5, 0:64].set(beta_food)
    prm = prm.at[6, :].set(b_dec1)
    uf = _k2(acc_eats, acc_rev, W_dec1, prm)

    i0 = edge_label_index[0].astype(jnp.int32)
    i1 = edge_label_index[1].astype(jnp.int32) + N
    zpad = jnp.zeros((LP - NLBL,), jnp.int32)
    idxs = jnp.concatenate([i0, zpad, i1, zpad])

    wb = jnp.concatenate([W_dec2[:, 0],
                          jnp.broadcast_to(b_dec2[0], (16,))])

    scores = _dec_call(uf, idxs, wb)
    return scores[:NLBL]
